# Initial kernel scaffold; baseline (speedup 1.0000x reference)
import jax
import jax.numpy as jnp
from jax.experimental import pallas as pl


def kernel(center, outside, all_vocabs, W_center, W_outside):
    raise NotImplementedError("write your pallas kernel here")


# speedup vs baseline: 44.7933x; 44.7933x over previous
import functools

import jax
import jax.numpy as jnp
from jax import lax
from jax.experimental import pallas as pl
from jax.experimental.pallas import tpu as pltpu
from jax.experimental.pallas import tpu_sc as plsc

VOC = 1000
EMB = 64
B = 1024
VP = 1024
NC, NS, L = 2, 16, 16
NW = NC * NS
RPT = B // NW


def _scores_body(center_ref, outside_ref, wc_ref, wot_ref, e_ref, top_ref):
    ids_c = center_ref[...]
    ids_o = outside_ref[...]
    iota_w = lax.broadcasted_iota(jnp.int32, (B, VP), 1)
    oh_c = (ids_c == iota_w).astype(jnp.float32)
    ce = lax.dot_general(
        oh_c, wc_ref[...], (((1,), (0,)), ((), ())),
        preferred_element_type=jnp.float32,
        precision=lax.Precision.HIGHEST)
    scores = lax.dot_general(
        ce, wot_ref[...], (((1,), (0,)), ((), ())),
        preferred_element_type=jnp.float32,
        precision=lax.Precision.HIGHEST)
    e_ref[...] = jnp.exp(scores)
    oh_o = (ids_o == iota_w).astype(jnp.float32)
    top_ref[0, 0] = jnp.sum(oh_o * scores)


_scores_call = pl.pallas_call(
    _scores_body,
    out_shape=[
        jax.ShapeDtypeStruct((B, VP), jnp.float32),
        jax.ShapeDtypeStruct((1, 1), jnp.float32),
    ],
    out_specs=[
        pl.BlockSpec(memory_space=pltpu.VMEM),
        pl.BlockSpec(memory_space=pltpu.SMEM),
    ],
)


def _sc_gather_sum_body(avt_hbm, eflat_hbm, out_hbm, av_v, e_v, out_v):
    wid = lax.axis_index("s") * NC + lax.axis_index("c")
    pltpu.sync_copy(avt_hbm.at[wid], av_v)
    pltpu.sync_copy(eflat_hbm.at[pl.ds(wid * (RPT * VP), RPT * VP)], e_v)
    for g in range(RPT // L):
        lane_off = (g * L + lax.iota(jnp.int32, L)) * VP

        def body(v, acc):
            idx = av_v[v, pl.ds(g * L, L)] + lane_off
            return acc + plsc.load_gather(e_v, [idx])

        acc = lax.fori_loop(0, VOC, body, jnp.zeros((L,), jnp.float32))
        out_v[pl.ds(g * L, L)] = acc
    pltpu.sync_copy(out_v, out_hbm.at[pl.ds(wid * RPT, RPT)])


@functools.cache
def _sc_gather_sum():
    return pl.kernel(
        _sc_gather_sum_body,
        out_type=jax.ShapeDtypeStruct((B,), jnp.float32),
        mesh=plsc.VectorSubcoreMesh(core_axis_name="c", subcore_axis_name="s",
                                    num_cores=NC, num_subcores=NS),
        scratch_types=[
            pltpu.VMEM((VOC, RPT), jnp.int32),
            pltpu.VMEM((RPT * VP,), jnp.float32),
            pltpu.VMEM((RPT,), jnp.float32),
        ],
        compiler_params=pltpu.CompilerParams(needs_layout_passes=False,
                                             use_tc_tiling_on_sc=False),
    )


def _loss_body(lower_ref, top_ref, out_ref):
    out_ref[0, 0] = (jnp.sum(jnp.log(lower_ref[...])) - top_ref[0, 0]) / B


_loss_call = pl.pallas_call(
    _loss_body,
    in_specs=[
        pl.BlockSpec(memory_space=pltpu.VMEM),
        pl.BlockSpec(memory_space=pltpu.SMEM),
    ],
    out_shape=jax.ShapeDtypeStruct((1, 1), jnp.float32),
    out_specs=pl.BlockSpec(memory_space=pltpu.SMEM),
)


def kernel(center, outside, all_vocabs, W_center, W_outside):
    center = center.reshape(B, 1).astype(jnp.int32)
    outside = outside.reshape(B, 1).astype(jnp.int32)
    av = all_vocabs.astype(jnp.int32)
    wc = jnp.pad(W_center.astype(jnp.float32), ((0, VP - VOC), (0, 0)))
    wot = jnp.pad(W_outside.astype(jnp.float32),
                  ((0, VP - VOC), (0, 0))).T
    e_mat, top_sum = _scores_call(center, outside, wc, wot)
    avt = av.T.reshape(VOC, NW, RPT).transpose(1, 0, 2)
    lower = _sc_gather_sum()(avt, e_mat.reshape(B * VP))
    loss = _loss_call(lower.reshape(8, 128), top_sum)
    return loss[0, 0]


# speedup vs baseline: 66.6952x; 1.4890x over previous
import functools

import jax
import jax.numpy as jnp
from jax import lax
from jax.experimental import pallas as pl
from jax.experimental.pallas import tpu as pltpu
from jax.experimental.pallas import tpu_sc as plsc

VOC = 1000
EMB = 64
B = 1024
VP = 1024
NC, NS, L = 2, 16, 16
NW = NC * NS
RPT = B // NW


def _expg_body(wc_ref, wot_ref, eg_ref):
    g = lax.dot_general(
        wc_ref[...], wot_ref[...], (((1,), (0,)), ((), ())),
        preferred_element_type=jnp.float32,
        precision=lax.Precision.HIGHEST)
    eg_ref[...] = jnp.exp(g)


_expg_call = pl.pallas_call(
    _expg_body,
    out_shape=jax.ShapeDtypeStruct((VOC, VP), jnp.float32),
)


def _sc_body(center_hbm, outside_hbm, av_hbm, eg_hbm,
             lower_hbm, top_hbm,
             center_v, outside_v, av_v, rows_v, lower_v, top_v, sem):
    wid = lax.axis_index("s") * NC + lax.axis_index("c")
    base = wid * RPT
    pltpu.sync_copy(center_hbm.at[pl.ds(base, RPT)], center_v)
    pltpu.sync_copy(outside_hbm.at[pl.ds(base, RPT)], outside_v)
    pltpu.sync_copy(av_hbm.at[pl.ds(base * VOC, RPT * VOC)], av_v)
    pltpu.async_copy(eg_hbm.at[center_v], rows_v, sem).wait()
    for g in range(RPT // L):
        lanes = g * L + lax.iota(jnp.int32, L)
        lanes_voc = lanes * VOC

        def body(v, acc):
            avv = plsc.load_gather(av_v, [lanes_voc + v])
            return acc + plsc.load_gather(rows_v, [lanes, avv])

        acc = lax.fori_loop(0, VOC, body, jnp.zeros((L,), jnp.float32))
        lower_v[pl.ds(g * L, L)] = acc
        ov = outside_v[pl.ds(g * L, L)]
        top_v[pl.ds(g * L, L)] = plsc.load_gather(rows_v, [lanes, ov])
    pltpu.sync_copy(lower_v, lower_hbm.at[pl.ds(base, RPT)])
    pltpu.sync_copy(top_v, top_hbm.at[pl.ds(base, RPT)])


@functools.cache
def _sc_call():
    return pl.kernel(
        _sc_body,
        out_type=[jax.ShapeDtypeStruct((B,), jnp.float32),
                  jax.ShapeDtypeStruct((B,), jnp.float32)],
        mesh=plsc.VectorSubcoreMesh(core_axis_name="c", subcore_axis_name="s",
                                    num_cores=NC, num_subcores=NS),
        scratch_types=[
            pltpu.VMEM((RPT,), jnp.int32),
            pltpu.VMEM((RPT,), jnp.int32),
            pltpu.VMEM((RPT * VOC,), jnp.int32),
            pltpu.VMEM((RPT, VP), jnp.float32),
            pltpu.VMEM((RPT,), jnp.float32),
            pltpu.VMEM((RPT,), jnp.float32),
            pltpu.SemaphoreType.DMA,
        ],
        compiler_params=pltpu.CompilerParams(needs_layout_passes=False,
                                             use_tc_tiling_on_sc=False),
    )


def _loss_body(lower_ref, top_ref, out_ref):
    out_ref[0, 0] = jnp.sum(jnp.log(lower_ref[...])
                            - jnp.log(top_ref[...])) / B


_loss_call = pl.pallas_call(
    _loss_body,
    out_shape=jax.ShapeDtypeStruct((1, 1), jnp.float32),
    out_specs=pl.BlockSpec(memory_space=pltpu.SMEM),
)


def kernel(center, outside, all_vocabs, W_center, W_outside):
    center = center.reshape(B).astype(jnp.int32)
    outside = outside.reshape(B).astype(jnp.int32)
    av = all_vocabs.astype(jnp.int32).reshape(B * VOC)
    wc = W_center.astype(jnp.float32)
    wot = jnp.pad(W_outside.astype(jnp.float32),
                  ((0, VP - VOC), (0, 0))).T
    eg = _expg_call(wc, wot)
    lower, top = _sc_call()(center, outside, av, eg)
    loss = _loss_call(lower.reshape(8, 128), top.reshape(8, 128))
    return loss[0, 0]


# speedup vs baseline: 74.8383x; 1.1221x over previous
import functools

import jax
import jax.numpy as jnp
from jax import lax
from jax.experimental import pallas as pl
from jax.experimental.pallas import tpu as pltpu
from jax.experimental.pallas import tpu_sc as plsc

VOC = 1000
EMB = 64
B = 1024
VP = 1024
NC, NS, L = 2, 16, 16
NW = NC * NS
RPT = B // NW


def _expg_body(wc_ref, wot_ref, eg_ref):
    g = lax.dot_general(
        wc_ref[...], wot_ref[...], (((1,), (0,)), ((), ())),
        preferred_element_type=jnp.float32,
        precision=lax.Precision.HIGHEST)
    eg_ref[...] = jnp.exp(g)


_expg_call = pl.pallas_call(
    _expg_body,
    out_shape=jax.ShapeDtypeStruct((VOC, VP), jnp.float32),
)


def _sc_body(center_hbm, outside_hbm, av_hbm, eg_hbm,
             lower_hbm, top_hbm,
             center_v, outside_v, av_v, rows_v, lower_v, top_v,
             sem_rows, sem_av):
    wid = lax.axis_index("s") * NC + lax.axis_index("c")
    base = wid * RPT
    pltpu.sync_copy(center_hbm.at[pl.ds(base, RPT)], center_v)
    pltpu.sync_copy(outside_hbm.at[pl.ds(base, RPT)], outside_v)
    av_cp = pltpu.async_copy(
        av_hbm.at[pl.ds(base * VOC, RPT * VOC)], av_v, sem_av)
    rows_cp = pltpu.async_copy(eg_hbm.at[center_v], rows_v, sem_rows)
    av_cp.wait()
    rows_cp.wait()
    for g in range(RPT // L):
        lanes = g * L + lax.iota(jnp.int32, L)
        lanes_voc = lanes * VOC

        @plsc.parallel_loop(0, VOC, unroll=8,
                            carry=jnp.zeros((L,), jnp.float32))
        def acc(v, acc_in):
            avv = plsc.load_gather(av_v, [lanes_voc + v])
            return acc_in + plsc.load_gather(rows_v, [lanes, avv])

        lower_v[pl.ds(g * L, L)] = acc
        ov = outside_v[pl.ds(g * L, L)]
        top_v[pl.ds(g * L, L)] = plsc.load_gather(rows_v, [lanes, ov])
    pltpu.sync_copy(lower_v, lower_hbm.at[pl.ds(base, RPT)])
    pltpu.sync_copy(top_v, top_hbm.at[pl.ds(base, RPT)])


@functools.cache
def _sc_call():
    return pl.kernel(
        _sc_body,
        out_type=[jax.ShapeDtypeStruct((B,), jnp.float32),
                  jax.ShapeDtypeStruct((B,), jnp.float32)],
        mesh=plsc.VectorSubcoreMesh(core_axis_name="c", subcore_axis_name="s",
                                    num_cores=NC, num_subcores=NS),
        scratch_types=[
            pltpu.VMEM((RPT,), jnp.int32),
            pltpu.VMEM((RPT,), jnp.int32),
            pltpu.VMEM((RPT * VOC,), jnp.int32),
            pltpu.VMEM((RPT, VP), jnp.float32),
            pltpu.VMEM((RPT,), jnp.float32),
            pltpu.VMEM((RPT,), jnp.float32),
            pltpu.SemaphoreType.DMA,
            pltpu.SemaphoreType.DMA,
        ],
        compiler_params=pltpu.CompilerParams(needs_layout_passes=False,
                                             use_tc_tiling_on_sc=False),
    )


def _loss_body(lower_ref, top_ref, out_ref):
    out_ref[0, 0] = jnp.sum(jnp.log(lower_ref[...])
                            - jnp.log(top_ref[...])) / B


_loss_call = pl.pallas_call(
    _loss_body,
    out_shape=jax.ShapeDtypeStruct((1, 1), jnp.float32),
    out_specs=pl.BlockSpec(memory_space=pltpu.SMEM),
)


def kernel(center, outside, all_vocabs, W_center, W_outside):
    center = center.reshape(B).astype(jnp.int32)
    outside = outside.reshape(B).astype(jnp.int32)
    av = all_vocabs.astype(jnp.int32).reshape(B * VOC)
    wc = W_center.astype(jnp.float32)
    wot = jnp.pad(W_outside.astype(jnp.float32),
                  ((0, VP - VOC), (0, 0))).T
    eg = _expg_call(wc, wot)
    lower, top = _sc_call()(center, outside, av, eg)
    loss = _loss_call(lower.reshape(8, 128), top.reshape(8, 128))
    return loss[0, 0]
